# baseline (speedup 1.0000x reference)
import functools

import jax
import jax.numpy as jnp
from jax import lax
from jax.experimental import pallas as pl
from jax.experimental.pallas import tpu as pltpu
from jax.experimental.pallas import tpu_sc as plsc

N = 10000
E = 320000
IN_DIM = 128
OUT_DIM = 64
NUM_HEADS = 4

NC = 2
NS = 16
NP = 10240
EPT = E // NS
K = 80
KB = K // 16
NCHUNK = EPT // K
G2 = NCHUNK // 2
RPT = NP // NS
RK = 80
RFULL = RPT // RK

BN = 640
NBLK = NP // BN


def _tc_body(h_ref, wc_ref, as_ref, ad_ref, z_ref, s_ref, d_ref):
    z = jnp.dot(h_ref[...], wc_ref[...], preferred_element_type=jnp.float32)
    z_ref[...] = z
    s_ref[...] = jnp.dot(z, as_ref[...], preferred_element_type=jnp.float32)[:, :2]
    d_ref[...] = jnp.dot(z, ad_ref[...], preferred_element_type=jnp.float32)[:, :2]


_tc_call = pl.pallas_call(
    _tc_body,
    grid=(NBLK, NC),
    in_specs=[
        pl.BlockSpec((BN, IN_DIM), lambda nb, c: (nb, 0)),
        pl.BlockSpec((IN_DIM, 128), lambda nb, c: (0, c)),
        pl.BlockSpec((128, 16), lambda nb, c: (c, 0)),
        pl.BlockSpec((128, 16), lambda nb, c: (c, 0)),
    ],
    out_specs=[
        pl.BlockSpec((BN, 128), lambda nb, c: (c * NBLK + nb, 0)),
        pl.BlockSpec((BN, 2), lambda nb, c: (c * NBLK + nb, 0)),
        pl.BlockSpec((BN, 2), lambda nb, c: (c * NBLK + nb, 0)),
    ],
    out_shape=[
        jax.ShapeDtypeStruct((NC * NP, 128), jnp.float32),
        jax.ShapeDtypeStruct((NC * NP, 2), jnp.float32),
        jax.ShapeDtypeStruct((NC * NP, 2), jnp.float32),
    ],
)


@functools.partial(
    pl.kernel,
    mesh=plsc.VectorSubcoreMesh(core_axis_name="c", subcore_axis_name="s"),
    compiler_params=pltpu.CompilerParams(needs_layout_passes=False),
    out_type=jax.ShapeDtypeStruct((NP, NUM_HEADS * OUT_DIM), jnp.float32),
    scratch_types=[
        pltpu.VMEM((2, K), jnp.int32),
        pltpu.VMEM((2, K), jnp.int32),
        pltpu.VMEM((2, K), jnp.int32),
        pltpu.VMEM((2, 32), jnp.int32),
        pltpu.VMEM((2, 32), jnp.int32),
        pltpu.VMEM((2, 16), jnp.int32),
        pltpu.VMEM((2, 2, K), jnp.int32),
        pltpu.VMEM((2, 2, K), jnp.float32),
        pltpu.VMEM((2, K, 128), jnp.float32),
        pltpu.VMEM((N,), jnp.float32),
        pltpu.VMEM((N,), jnp.float32),
        pltpu.VMEM((2 * (2 * RK + 24),), jnp.float32),
        pltpu.VMEM_SHARED((NP, 128), jnp.float32),
        pltpu.VMEM_SHARED((2 * NP,), jnp.float32),
        pltpu.SemaphoreType.DMA,
        pltpu.SemaphoreType.DMA,
        pltpu.SemaphoreType.DMA,
        pltpu.SemaphoreType.DMA,
        pltpu.SemaphoreType.DMA,
        pltpu.SemaphoreType.DMA,
        pltpu.SemaphoreType.DMA,
        pltpu.SemaphoreType.DMA,
        pltpu.SemaphoreType.DMA,
        pltpu.SemaphoreType.DMA,
    ],
)
def _sc_edge_kernel(src_hbm, dst_hbm, z_hbm, atts_hbm, attd_hbm, out_hbm,
                    src_v, dst_v, srcz_v, dst2a_v, dst2b_v, dst2c_v, dstd_v,
                    w_v, zbuf_v,
                    atts_vm, attd_vm, dbuf_v, num_sh, den_sh,
                    isem0, isem1, gsem0, gsem1,
                    ssem0, ssem1, dsem00, dsem01, dsem10, dsem11):
    c = lax.axis_index("c")
    sid = lax.axis_index("s")
    zero16 = jnp.zeros((16,), jnp.float32)
    gsems = (gsem0, gsem1)
    ssems = (ssem0, ssem1)
    dsems = ((dsem00, dsem01), (dsem10, dsem11))
    isems = (isem0, isem1)

    abase = pl.multiple_of(c * N, 8)
    pltpu.sync_copy(atts_hbm.at[pl.ds(abase, N)], atts_vm)
    pltpu.sync_copy(attd_hbm.at[pl.ds(abase, N)], attd_vm)

    for m in range(K):
        for i in range(8):
            zbuf_v[0, m, pl.ds(i * 16, 16)] = zero16

    rbase = sid * RPT
    for g in range(RPT // K):
        row0 = pl.multiple_of(rbase + g * K, 8)
        pltpu.sync_copy(zbuf_v.at[0], num_sh.at[pl.ds(row0, K)])
    for g in range(2 * RPT // 128):
        d0 = pl.multiple_of(2 * rbase + g * 128, 8)
        pltpu.sync_copy(zbuf_v.at[0, 0], den_sh.at[pl.ds(d0, 128)])

    plsc.subcore_barrier()

    ebase = sid * EPT
    zoff = c * NP

    def _base(g):
        return pl.multiple_of(ebase + g * K, 8)

    def _issue_idx(p, base):
        pltpu.async_copy(src_hbm.at[pl.ds(base, K)], src_v.at[p], isems[p])
        pltpu.async_copy(dst_hbm.at[pl.ds(base, K)], dst_v.at[p], isems[p])

    def _drain_idx(p, base):
        pltpu.make_async_copy(src_hbm.at[pl.ds(base, K)], src_v.at[p],
                              isems[p]).wait()
        pltpu.make_async_copy(dst_hbm.at[pl.ds(base, K)], dst_v.at[p],
                              isems[p]).wait()

    def _make_srcz(p):
        for kk in range(KB):
            srcz_v[p, pl.ds(kk * 16, 16)] = (
                src_v[p, pl.ds(kk * 16, 16)] + zoff)

    def _issue_gather(p):
        return pltpu.async_copy(z_hbm.at[srcz_v.at[p]], zbuf_v.at[p],
                                gsems[p])

    def _drain_gather(p):
        pltpu.make_async_copy(z_hbm.at[srcz_v.at[p]], zbuf_v.at[p],
                              gsems[p]).wait()

    def _drain_scat(p):
        pltpu.make_async_copy(zbuf_v.at[p, pl.ds(0, 32)],
                              num_sh.at[dst2a_v.at[p]], ssems[p]).wait()
        pltpu.make_async_copy(zbuf_v.at[p, pl.ds(32, 32)],
                              num_sh.at[dst2b_v.at[p]], ssems[p]).wait()
        pltpu.make_async_copy(zbuf_v.at[p, pl.ds(64, 16)],
                              num_sh.at[dst2c_v.at[p]], ssems[p]).wait()

    def _drain_den(p):
        pltpu.make_async_copy(w_v.at[p, 0], den_sh.at[dstd_v.at[p, 0]],
                              dsems[p][0]).wait()
        pltpu.make_async_copy(w_v.at[p, 1], den_sh.at[dstd_v.at[p, 1]],
                              dsems[p][1]).wait()

    def _weight_blocks(p):
        for kk in range(KB):
            k0 = kk * 16
            src16 = src_v[p, pl.ds(k0, 16)]
            dst16 = dst_v[p, pl.ds(k0, 16)]
            gs = plsc.load_gather(atts_vm, [src16])
            gd = plsc.load_gather(attd_vm, [dst16])
            s0, s1 = plsc.unpack(plsc.bitcast(gs, jnp.bfloat16),
                                 format=plsc.PackFormat.INTERLEAVED)
            d0, d1 = plsc.unpack(plsc.bitcast(gd, jnp.bfloat16),
                                 format=plsc.PackFormat.INTERLEAVED)
            e0 = s0 + d0
            e1 = s1 + d1
            e0 = jnp.where(e0 > 0, e0, 0.01 * e0)
            e1 = jnp.where(e1 > 0, e1, 0.01 * e1)
            w_v[p, 0, pl.ds(k0, 16)] = jnp.exp(e0)
            w_v[p, 1, pl.ds(k0, 16)] = jnp.exp(e1)
            if k0 < 32:
                dst2a_v[p, pl.ds(k0, 16)] = dst16
            elif k0 < 64:
                dst2b_v[p, pl.ds(k0 - 32, 16)] = dst16
            else:
                dst2c_v[p, pl.ds(k0 - 64, 16)] = dst16
            dstd_v[p, 0, pl.ds(k0, 16)] = dst16 * 2
            dstd_v[p, 1, pl.ds(k0, 16)] = dst16 * 2 + 1
        pltpu.async_copy(w_v.at[p, 0], den_sh.at[dstd_v.at[p, 0]],
                         dsems[p][0], add=True)
        pltpu.async_copy(w_v.at[p, 1], den_sh.at[dstd_v.at[p, 1]],
                         dsems[p][1], add=True)

    def _apply_weights(p):
        for kk in range(KB):
            k0 = kk * 16
            wv0 = w_v[p, 0, pl.ds(k0, 16)]
            wv1 = w_v[p, 1, pl.ds(k0, 16)]
            for m in range(16):
                a = wv0[m]
                bb = wv1[m]
                for i in range(4):
                    zbuf_v[p, k0 + m, pl.ds(i * 16, 16)] = (
                        zbuf_v[p, k0 + m, pl.ds(i * 16, 16)] * a)
                for i in range(4, 8):
                    zbuf_v[p, k0 + m, pl.ds(i * 16, 16)] = (
                        zbuf_v[p, k0 + m, pl.ds(i * 16, 16)] * bb)
            if kk == 1:
                pltpu.async_copy(zbuf_v.at[p, pl.ds(0, 32)],
                                 num_sh.at[dst2a_v.at[p]], ssems[p], add=True)
            elif kk == 3:
                pltpu.async_copy(zbuf_v.at[p, pl.ds(32, 32)],
                                 num_sh.at[dst2b_v.at[p]], ssems[p], add=True)

    def _issue_scat(p):
        pltpu.async_copy(zbuf_v.at[p, pl.ds(64, 16)],
                         num_sh.at[dst2c_v.at[p]], ssems[p], add=True)

    _issue_idx(0, _base(0))
    _issue_idx(1, _base(1))
    _drain_idx(0, _base(0))
    _make_srcz(0)
    _issue_gather(0)

    def _body(t, cc):
        g0 = 2 * t

        @pl.when(t > 0)
        def _():
            _drain_den(0)
        _drain_idx(1, _base(g0 + 1))
        _make_srcz(1)
        _weight_blocks(0)

        @pl.when(t > 0)
        def _():
            _drain_scat(1)
        _issue_gather(1)

        @pl.when(g0 + 2 < NCHUNK)
        def _():
            _issue_idx(0, _base(g0 + 2))
        _drain_gather(0)
        _apply_weights(0)
        _issue_scat(0)

        @pl.when(t > 0)
        def _():
            _drain_den(1)
        _weight_blocks(1)

        @pl.when(g0 + 2 < NCHUNK)
        def _():
            _drain_idx(0, _base(g0 + 2))
            _make_srcz(0)
            _drain_scat(0)
            _issue_gather(0)

        @pl.when(g0 + 3 < NCHUNK)
        def _():
            _issue_idx(1, _base(g0 + 3))
        _drain_gather(1)
        _apply_weights(1)
        _issue_scat(1)

        return cc

    lax.fori_loop(0, G2, _body, 0)

    _drain_scat(0)
    _drain_scat(1)
    _drain_den(0)
    _drain_den(1)

    plsc.subcore_barrier()

    ocol = pl.multiple_of(c * 128, 128)

    def _f_issue_in(p, gg):
        row0 = pl.multiple_of(rbase + gg * RK, 8)
        pltpu.async_copy(num_sh.at[pl.ds(row0, RK)], zbuf_v.at[p], isems[p])
        pltpu.async_copy(den_sh.at[pl.ds(2 * row0, 2 * RK)],
                         dbuf_v.at[pl.ds(p * (2 * RK + 24), 2 * RK)],
                         dsems[p][0])

    def _f_drain_in(p, gg):
        row0 = pl.multiple_of(rbase + gg * RK, 8)
        pltpu.make_async_copy(num_sh.at[pl.ds(row0, RK)], zbuf_v.at[p],
                              isems[p]).wait()
        pltpu.make_async_copy(den_sh.at[pl.ds(2 * row0, 2 * RK)],
                              dbuf_v.at[pl.ds(p * (2 * RK + 24), 2 * RK)],
                              dsems[p][0]).wait()

    def _f_issue_out(p, gg):
        row0 = pl.multiple_of(rbase + gg * RK, 8)
        pltpu.async_copy(zbuf_v.at[p],
                         out_hbm.at[pl.ds(row0, RK), pl.ds(ocol, 128)],
                         gsems[p])

    def _f_drain_out(p, gg):
        row0 = pl.multiple_of(rbase + gg * RK, 8)
        pltpu.make_async_copy(zbuf_v.at[p],
                              out_hbm.at[pl.ds(row0, RK), pl.ds(ocol, 128)],
                              gsems[p]).wait()

    def _f_compute(p):
        doff = p * (2 * RK + 24)
        for i in range(2 * RK // 16):
            dv = dbuf_v[pl.ds(doff + i * 16, 16)]
            dbuf_v[pl.ds(doff + i * 16, 16)] = jnp.where(dv != 0.0, 1.0 / dv,
                                                         0.0)
        for m in range(RK):
            iv = dbuf_v[pl.ds(doff + 2 * m, 16)]
            i0 = iv[0]
            i1 = iv[1]
            for i in range(4):
                zbuf_v[p, m, pl.ds(i * 16, 16)] = (
                    zbuf_v[p, m, pl.ds(i * 16, 16)] * i0)
            for i in range(4, 8):
                zbuf_v[p, m, pl.ds(i * 16, 16)] = (
                    zbuf_v[p, m, pl.ds(i * 16, 16)] * i1)

    _f_issue_in(0, 0)

    def _f_body(u, cc):
        gg0 = 2 * u
        _f_drain_in(0, gg0)

        @pl.when(u > 0)
        def _():
            _f_drain_out(1, gg0 - 1)
        _f_issue_in(1, gg0 + 1)
        _f_compute(0)
        _f_issue_out(0, gg0)
        _f_drain_in(1, gg0 + 1)

        @pl.when(gg0 + 2 < RFULL)
        def _():
            _f_drain_out(0, gg0)
            _f_issue_in(0, gg0 + 2)
        _f_compute(1)
        _f_issue_out(1, gg0 + 1)
        return cc

    lax.fori_loop(0, RFULL // 2, _f_body, 0)
    _f_drain_out(0, RFULL - 2)
    _f_drain_out(1, RFULL - 1)


def kernel(h, s, edge_index, W, A):
    del s
    src = edge_index[0]
    dst = edge_index[1]
    wc = jnp.transpose(W, (1, 0, 2)).reshape(IN_DIM, NUM_HEADS * OUT_DIM)
    asmat = jnp.zeros((NC * 128, 16), jnp.float32)
    admat = jnp.zeros((NC * 128, 16), jnp.float32)
    for i in range(NUM_HEADS):
        cblk, j = divmod(i, 2)
        rows = slice(cblk * 128 + j * OUT_DIM, cblk * 128 + (j + 1) * OUT_DIM)
        asmat = asmat.at[rows, j].set(A[i, :OUT_DIM, 0])
        admat = admat.at[rows, j].set(A[i, OUT_DIM:, 0])
    z2, atts, attd = _tc_call(h, wc, asmat, admat)
    atts1 = jax.lax.bitcast_convert_type(
        atts.reshape(NC, NP, 2)[:, :N, :].astype(jnp.bfloat16),
        jnp.float32).reshape(-1)
    attd1 = jax.lax.bitcast_convert_type(
        attd.reshape(NC, NP, 2)[:, :N, :].astype(jnp.bfloat16),
        jnp.float32).reshape(-1)
    out2 = _sc_edge_kernel(src, dst, z2, atts1, attd1)
    return out2[:N]


# speedup vs baseline: 1.0009x; 1.0009x over previous
import functools

import jax
import jax.numpy as jnp
from jax import lax
from jax.experimental import pallas as pl
from jax.experimental.pallas import tpu as pltpu
from jax.experimental.pallas import tpu_sc as plsc

N = 10000
E = 320000
IN_DIM = 128
OUT_DIM = 64
NUM_HEADS = 4

NC = 2
NS = 16
NP = 10240
EPT = E // NS
K = 80
KB = K // 16
NCHUNK = EPT // K
G2 = NCHUNK // 2
RPT = NP // NS
RK = 80
RFULL = RPT // RK

BN = 640
NBLK = NP // BN


def _tc_body(h_ref, wc_ref, as_ref, ad_ref, z_ref, s_ref, d_ref):
    z = jnp.dot(h_ref[...], wc_ref[...], preferred_element_type=jnp.float32)
    z_ref[...] = z
    s_ref[...] = jnp.dot(z, as_ref[...], preferred_element_type=jnp.float32)[:, :2]
    d_ref[...] = jnp.dot(z, ad_ref[...], preferred_element_type=jnp.float32)[:, :2]


_tc_call = pl.pallas_call(
    _tc_body,
    grid=(NBLK, NC),
    in_specs=[
        pl.BlockSpec((BN, IN_DIM), lambda nb, c: (nb, 0)),
        pl.BlockSpec((IN_DIM, 128), lambda nb, c: (0, c)),
        pl.BlockSpec((128, 16), lambda nb, c: (c, 0)),
        pl.BlockSpec((128, 16), lambda nb, c: (c, 0)),
    ],
    out_specs=[
        pl.BlockSpec((BN, 128), lambda nb, c: (c * NBLK + nb, 0)),
        pl.BlockSpec((BN, 2), lambda nb, c: (c * NBLK + nb, 0)),
        pl.BlockSpec((BN, 2), lambda nb, c: (c * NBLK + nb, 0)),
    ],
    out_shape=[
        jax.ShapeDtypeStruct((NC * NP, 128), jnp.float32),
        jax.ShapeDtypeStruct((NC * NP, 2), jnp.float32),
        jax.ShapeDtypeStruct((NC * NP, 2), jnp.float32),
    ],
)


@functools.partial(
    pl.kernel,
    mesh=plsc.VectorSubcoreMesh(core_axis_name="c", subcore_axis_name="s"),
    compiler_params=pltpu.CompilerParams(needs_layout_passes=False),
    out_type=jax.ShapeDtypeStruct((NP, NUM_HEADS * OUT_DIM), jnp.float32),
    scratch_types=[
        pltpu.VMEM((2, K), jnp.int32),
        pltpu.VMEM((2, K), jnp.int32),
        pltpu.VMEM((2, K), jnp.int32),
        pltpu.VMEM((2, 32), jnp.int32),
        pltpu.VMEM((2, 32), jnp.int32),
        pltpu.VMEM((2, 16), jnp.int32),
        pltpu.VMEM((2, 2, K), jnp.int32),
        pltpu.VMEM((2, 2, K), jnp.float32),
        pltpu.VMEM((2, K, 128), jnp.float32),
        pltpu.VMEM((N,), jnp.float32),
        pltpu.VMEM((N,), jnp.float32),
        pltpu.VMEM((2 * RK + 16,), jnp.float32),
        pltpu.VMEM_SHARED((NP, 128), jnp.float32),
        pltpu.VMEM_SHARED((2 * NP,), jnp.float32),
        pltpu.SemaphoreType.DMA,
        pltpu.SemaphoreType.DMA,
        pltpu.SemaphoreType.DMA,
        pltpu.SemaphoreType.DMA,
        pltpu.SemaphoreType.DMA,
        pltpu.SemaphoreType.DMA,
        pltpu.SemaphoreType.DMA,
        pltpu.SemaphoreType.DMA,
        pltpu.SemaphoreType.DMA,
        pltpu.SemaphoreType.DMA,
    ],
)
def _sc_edge_kernel(src_hbm, dst_hbm, z_hbm, atts_hbm, attd_hbm, out_hbm,
                    src_v, dst_v, srcz_v, dst2a_v, dst2b_v, dst2c_v, dstd_v,
                    w_v, zbuf_v,
                    atts_vm, attd_vm, dbuf_v, num_sh, den_sh,
                    isem0, isem1, gsem0, gsem1,
                    ssem0, ssem1, dsem00, dsem01, dsem10, dsem11):
    c = lax.axis_index("c")
    sid = lax.axis_index("s")
    zero16 = jnp.zeros((16,), jnp.float32)
    gsems = (gsem0, gsem1)
    ssems = (ssem0, ssem1)
    dsems = ((dsem00, dsem01), (dsem10, dsem11))
    isems = (isem0, isem1)

    abase = pl.multiple_of(c * N, 8)
    pltpu.sync_copy(atts_hbm.at[pl.ds(abase, N)], atts_vm)
    pltpu.sync_copy(attd_hbm.at[pl.ds(abase, N)], attd_vm)

    for m in range(K):
        for i in range(8):
            zbuf_v[0, m, pl.ds(i * 16, 16)] = zero16

    rbase = sid * RPT
    for g in range(RPT // K):
        row0 = pl.multiple_of(rbase + g * K, 8)
        pltpu.sync_copy(zbuf_v.at[0], num_sh.at[pl.ds(row0, K)])
    for g in range(2 * RPT // 128):
        d0 = pl.multiple_of(2 * rbase + g * 128, 8)
        pltpu.sync_copy(zbuf_v.at[0, 0], den_sh.at[pl.ds(d0, 128)])

    plsc.subcore_barrier()

    ebase = sid * EPT
    zoff = c * NP

    def _base(g):
        return pl.multiple_of(ebase + g * K, 8)

    def _issue_idx(p, base):
        pltpu.async_copy(src_hbm.at[pl.ds(base, K)], src_v.at[p], isems[p])
        pltpu.async_copy(dst_hbm.at[pl.ds(base, K)], dst_v.at[p], isems[p])

    def _drain_idx(p, base):
        pltpu.make_async_copy(src_hbm.at[pl.ds(base, K)], src_v.at[p],
                              isems[p]).wait()
        pltpu.make_async_copy(dst_hbm.at[pl.ds(base, K)], dst_v.at[p],
                              isems[p]).wait()

    def _make_srcz(p):
        for kk in range(KB):
            srcz_v[p, pl.ds(kk * 16, 16)] = (
                src_v[p, pl.ds(kk * 16, 16)] + zoff)

    def _issue_gather(p):
        return pltpu.async_copy(z_hbm.at[srcz_v.at[p]], zbuf_v.at[p],
                                gsems[p])

    def _drain_gather(p):
        pltpu.make_async_copy(z_hbm.at[srcz_v.at[p]], zbuf_v.at[p],
                              gsems[p]).wait()

    def _drain_scat(p):
        pltpu.make_async_copy(zbuf_v.at[p, pl.ds(0, 32)],
                              num_sh.at[dst2a_v.at[p]], ssems[p]).wait()
        pltpu.make_async_copy(zbuf_v.at[p, pl.ds(32, 32)],
                              num_sh.at[dst2b_v.at[p]], ssems[p]).wait()
        pltpu.make_async_copy(zbuf_v.at[p, pl.ds(64, 16)],
                              num_sh.at[dst2c_v.at[p]], ssems[p]).wait()

    def _drain_den(p):
        pltpu.make_async_copy(w_v.at[p, 0], den_sh.at[dstd_v.at[p, 0]],
                              dsems[p][0]).wait()
        pltpu.make_async_copy(w_v.at[p, 1], den_sh.at[dstd_v.at[p, 1]],
                              dsems[p][1]).wait()

    def _weight_blocks(p):
        for kk in range(KB):
            k0 = kk * 16
            src16 = src_v[p, pl.ds(k0, 16)]
            dst16 = dst_v[p, pl.ds(k0, 16)]
            gs = plsc.load_gather(atts_vm, [src16])
            gd = plsc.load_gather(attd_vm, [dst16])
            s0, s1 = plsc.unpack(plsc.bitcast(gs, jnp.bfloat16),
                                 format=plsc.PackFormat.INTERLEAVED)
            d0, d1 = plsc.unpack(plsc.bitcast(gd, jnp.bfloat16),
                                 format=plsc.PackFormat.INTERLEAVED)
            e0 = s0 + d0
            e1 = s1 + d1
            e0 = jnp.where(e0 > 0, e0, 0.01 * e0)
            e1 = jnp.where(e1 > 0, e1, 0.01 * e1)
            w_v[p, 0, pl.ds(k0, 16)] = jnp.exp(e0)
            w_v[p, 1, pl.ds(k0, 16)] = jnp.exp(e1)
            if k0 < 32:
                dst2a_v[p, pl.ds(k0, 16)] = dst16
            elif k0 < 64:
                dst2b_v[p, pl.ds(k0 - 32, 16)] = dst16
            else:
                dst2c_v[p, pl.ds(k0 - 64, 16)] = dst16
            dstd_v[p, 0, pl.ds(k0, 16)] = dst16 * 2
            dstd_v[p, 1, pl.ds(k0, 16)] = dst16 * 2 + 1
        pltpu.async_copy(w_v.at[p, 0], den_sh.at[dstd_v.at[p, 0]],
                         dsems[p][0], add=True)
        pltpu.async_copy(w_v.at[p, 1], den_sh.at[dstd_v.at[p, 1]],
                         dsems[p][1], add=True)

    def _apply_weights(p):
        for kk in range(KB):
            k0 = kk * 16
            wv0 = w_v[p, 0, pl.ds(k0, 16)]
            wv1 = w_v[p, 1, pl.ds(k0, 16)]
            for m in range(16):
                a = wv0[m]
                bb = wv1[m]
                for i in range(4):
                    zbuf_v[p, k0 + m, pl.ds(i * 16, 16)] = (
                        zbuf_v[p, k0 + m, pl.ds(i * 16, 16)] * a)
                for i in range(4, 8):
                    zbuf_v[p, k0 + m, pl.ds(i * 16, 16)] = (
                        zbuf_v[p, k0 + m, pl.ds(i * 16, 16)] * bb)
            if kk == 1:
                pltpu.async_copy(zbuf_v.at[p, pl.ds(0, 32)],
                                 num_sh.at[dst2a_v.at[p]], ssems[p], add=True)
            elif kk == 3:
                pltpu.async_copy(zbuf_v.at[p, pl.ds(32, 32)],
                                 num_sh.at[dst2b_v.at[p]], ssems[p], add=True)

    def _issue_scat(p):
        pltpu.async_copy(zbuf_v.at[p, pl.ds(64, 16)],
                         num_sh.at[dst2c_v.at[p]], ssems[p], add=True)

    _issue_idx(0, _base(0))
    _issue_idx(1, _base(1))
    _drain_idx(0, _base(0))
    _make_srcz(0)
    _issue_gather(0)

    def _body(t, cc):
        g0 = 2 * t

        @pl.when(t > 0)
        def _():
            _drain_den(0)
        _drain_idx(1, _base(g0 + 1))
        _make_srcz(1)
        _weight_blocks(0)

        @pl.when(t > 0)
        def _():
            _drain_scat(1)
        _issue_gather(1)

        @pl.when(g0 + 2 < NCHUNK)
        def _():
            _issue_idx(0, _base(g0 + 2))
        _drain_gather(0)
        _apply_weights(0)
        _issue_scat(0)

        @pl.when(t > 0)
        def _():
            _drain_den(1)
        _weight_blocks(1)

        @pl.when(g0 + 2 < NCHUNK)
        def _():
            _drain_idx(0, _base(g0 + 2))
            _make_srcz(0)
            _drain_scat(0)
            _issue_gather(0)

        @pl.when(g0 + 3 < NCHUNK)
        def _():
            _issue_idx(1, _base(g0 + 3))
        _drain_gather(1)
        _apply_weights(1)
        _issue_scat(1)

        return cc

    lax.fori_loop(0, G2, _body, 0)

    _drain_scat(0)
    _drain_scat(1)
    _drain_den(0)
    _drain_den(1)

    plsc.subcore_barrier()

    ocol = pl.multiple_of(c * 128, 128)

    def _rows(gg, cc):
        row0 = pl.multiple_of(rbase + gg * RK, 8)
        pltpu.sync_copy(num_sh.at[pl.ds(row0, RK)], zbuf_v.at[0])
        pltpu.sync_copy(den_sh.at[pl.ds(2 * row0, 2 * RK)],
                        dbuf_v.at[pl.ds(0, 2 * RK)])
        for i in range(2 * RK // 16):
            dv = dbuf_v[pl.ds(i * 16, 16)]
            dbuf_v[pl.ds(i * 16, 16)] = jnp.where(dv != 0.0, 1.0 / dv, 0.0)
        for m in range(RK):
            iv = dbuf_v[pl.ds(2 * m, 16)]
            i0 = iv[0]
            i1 = iv[1]
            for i in range(4):
                zbuf_v[0, m, pl.ds(i * 16, 16)] = (
                    zbuf_v[0, m, pl.ds(i * 16, 16)] * i0)
            for i in range(4, 8):
                zbuf_v[0, m, pl.ds(i * 16, 16)] = (
                    zbuf_v[0, m, pl.ds(i * 16, 16)] * i1)
        pltpu.sync_copy(zbuf_v.at[0],
                        out_hbm.at[pl.ds(row0, RK), pl.ds(ocol, 128)])
        return cc

    lax.fori_loop(0, RFULL, _rows, 0)


def kernel(h, s, edge_index, W, A):
    del s
    src = edge_index[0]
    dst = edge_index[1]
    wc = jnp.transpose(W, (1, 0, 2)).reshape(IN_DIM, NUM_HEADS * OUT_DIM)
    asmat = jnp.zeros((NC * 128, 16), jnp.float32)
    admat = jnp.zeros((NC * 128, 16), jnp.float32)
    for i in range(NUM_HEADS):
        cblk, j = divmod(i, 2)
        rows = slice(cblk * 128 + j * OUT_DIM, cblk * 128 + (j + 1) * OUT_DIM)
        asmat = asmat.at[rows, j].set(A[i, :OUT_DIM, 0])
        admat = admat.at[rows, j].set(A[i, OUT_DIM:, 0])
    z2, atts, attd = _tc_call(h, wc, asmat, admat)
    atts1 = jax.lax.bitcast_convert_type(
        atts.reshape(NC, NP, 2)[:, :N, :].astype(jnp.bfloat16),
        jnp.float32).reshape(-1)
    attd1 = jax.lax.bitcast_convert_type(
        attd.reshape(NC, NP, 2)[:, :N, :].astype(jnp.bfloat16),
        jnp.float32).reshape(-1)
    out2 = _sc_edge_kernel(src, dst, z2, atts1, attd1)
    return out2[:N]
